# baseline (speedup 1.0000x reference)
import functools

import jax
import jax.numpy as jnp
from jax import lax
from jax.experimental import pallas as pl
from jax.experimental.pallas import tpu as pltpu
from jax.experimental.pallas import tpu_sc as plsc

VOCAB = 100000
EMBED_DIM = 128
CTX_LEN = 200

_CHUNK = 8
_NCHUNKS = CTX_LEN // _CHUNK

_mesh = plsc.VectorSubcoreMesh(core_axis_name="c", subcore_axis_name="s")


@functools.partial(
    pl.kernel,
    mesh=_mesh,
    out_type=jax.ShapeDtypeStruct((CTX_LEN, EMBED_DIM), jnp.float32),
    scratch_types=[
        pltpu.VMEM((_CHUNK,), jnp.int32),
        pltpu.VMEM((_CHUNK, EMBED_DIM), jnp.float32),
        pltpu.SemaphoreType.DMA,
    ],
)
def _sc_gather(idx_hbm, table_hbm, out_hbm, idx_v, rows_v, sem):
    wid = lax.axis_index("s") * 2 + lax.axis_index("c")

    @pl.when(wid < _NCHUNKS)
    def _():
        base = wid * _CHUNK
        pltpu.sync_copy(idx_hbm.at[pl.ds(base, _CHUNK)], idx_v)
        pltpu.async_copy(table_hbm.at[idx_v], rows_v, sem).wait()
        pltpu.sync_copy(rows_v, out_hbm.at[pl.ds(base, _CHUNK)])


_BV = 8192


def _tc_matvec_body(rows_ref, w_ref, b_ref, out_ref, mean_ref):
    @pl.when(pl.program_id(0) == 0)
    def _():
        m = jnp.sum(rows_ref[...], axis=0, keepdims=True) * (1.0 / CTX_LEN)
        mean_ref[...] = m

    acc = jax.lax.dot_general(
        mean_ref[...], w_ref[...],
        (((1,), (1,)), ((), ())),
        preferred_element_type=jnp.float32,
    )
    out_ref[...] = acc + b_ref[...]


def kernel(context_words, embeddings, W, b):
    rows = _sc_gather(context_words, embeddings)

    nb = (VOCAB + _BV - 1) // _BV
    out2d = pl.pallas_call(
        _tc_matvec_body,
        grid=(nb,),
        in_specs=[
            pl.BlockSpec((CTX_LEN, EMBED_DIM), lambda i: (0, 0)),
            pl.BlockSpec((_BV, EMBED_DIM), lambda i: (i, 0)),
            pl.BlockSpec((1, _BV), lambda i: (0, i)),
        ],
        out_specs=pl.BlockSpec((1, _BV), lambda i: (0, i)),
        out_shape=jax.ShapeDtypeStruct((1, VOCAB), jnp.float32),
        scratch_shapes=[pltpu.VMEM((1, EMBED_DIM), jnp.float32)],
    )(rows, W, b.reshape(1, VOCAB))
    return out2d[0]
